# baseline (speedup 1.0000x reference)
import jax
import jax.numpy as jnp
from jax.experimental import pallas as pl
from jax.experimental.pallas import tpu as pltpu

_B, _D, _T = 16, 64, 576
_K = 1024
_BB = 8


def _vq_body(z_ref, cb_ref, out_ref, cc_s, c1_s, c2_s, c3_s):
    cb = cb_ref[...]

    @pl.when(pl.program_id(0) == 0)
    def _prep():
        cc_s[0] = jnp.sum(cb * cb, axis=1)
        c1 = cb.astype(jnp.bfloat16)
        e1 = cb - c1.astype(jnp.float32)
        c2 = e1.astype(jnp.bfloat16)
        e2 = e1 - c2.astype(jnp.float32)
        c1_s[...] = c1
        c2_s[...] = c2
        c3_s[...] = e2.astype(jnp.bfloat16)

    zb = jnp.transpose(z_ref[...], (0, 2, 1)).reshape(_BB * _T, _D)
    m2 = jax.lax.dot_general(zb, cb, (((1,), (1,)), ((), ())))
    zz = jnp.sum(zb * zb, axis=1, keepdims=True)
    dists = (zz - 2.0 * m2) + cc_s[0][None, :]
    idxs = jnp.argmin(dists, axis=1).astype(jnp.int32)
    kio = jax.lax.broadcasted_iota(jnp.int32, (_BB * _T, _K), 1)
    onehot = (kio == idxs[:, None]).astype(jnp.bfloat16)

    def dec(c_s):
        return jax.lax.dot_general(
            c_s[...], onehot, (((0,), (1,)), ((), ())),
            preferred_element_type=jnp.float32)

    q = (dec(c1_s) + dec(c2_s)) + dec(c3_s)
    for i in range(_BB):
        out_ref[i] = q[:, i * _T:(i + 1) * _T]


def kernel(z, codebook):
    return pl.pallas_call(
        _vq_body,
        grid=(_B // _BB,),
        in_specs=[
            pl.BlockSpec((_BB, _D, _T), lambda b: (b, 0, 0)),
            pl.BlockSpec((_K, _D), lambda b: (0, 0)),
        ],
        out_specs=pl.BlockSpec((_BB, _D, _T), lambda b: (b, 0, 0)),
        out_shape=jax.ShapeDtypeStruct((_B, _D, _T), jnp.float32),
        scratch_shapes=[
            pltpu.VMEM((1, _K), jnp.float32),
            pltpu.VMEM((_K, _D), jnp.bfloat16),
            pltpu.VMEM((_K, _D), jnp.bfloat16),
            pltpu.VMEM((_K, _D), jnp.bfloat16),
        ],
    )(z, codebook)


# speedup vs baseline: 1.4053x; 1.4053x over previous
import jax
import jax.numpy as jnp
from jax.experimental import pallas as pl
from jax.experimental.pallas import tpu as pltpu

_B, _D, _T = 16, 64, 576
_K = 1024
_BB = 4
_R = (_BB // 2) * _T


def _vq_body(z_ref, cb_ref, out_ref, cc_s, cs_s):
    cb = cb_ref[...]

    @pl.when(pl.program_id(0) == 0)
    def _prep():
        cc_s[0] = jnp.sum(cb * cb, axis=1)
        c1 = cb.astype(jnp.bfloat16)
        e1 = cb - c1.astype(jnp.float32)
        c2 = e1.astype(jnp.bfloat16)
        e2 = e1 - c2.astype(jnp.float32)
        cs_s[:, 0:_D] = c1
        cs_s[:, _D:2 * _D] = c2
        cs_s[:, 2 * _D:3 * _D] = e2.astype(jnp.bfloat16)

    zb = jnp.transpose(z_ref[...], (0, 2, 1)).reshape(_BB * _T, _D)
    m2 = jax.lax.dot_general(zb, cb, (((1,), (1,)), ((), ())))
    zz = jnp.sum(zb * zb, axis=1, keepdims=True)
    dists = (zz - 2.0 * m2) + cc_s[0][None, :]
    idxs = jnp.argmin(dists, axis=1).astype(jnp.int32)
    kio = jax.lax.broadcasted_iota(jnp.int32, (_BB * _T, _K), 1)
    onehot = (kio == idxs[:, None]).astype(jnp.bfloat16)

    qs = jax.lax.dot_general(
        cs_s[...], onehot, (((0,), (1,)), ((), ())),
        preferred_element_type=jnp.float32)
    q = (qs[0:_D] + qs[_D:2 * _D]) + qs[2 * _D:3 * _D]
    for i in range(_BB):
        out_ref[i] = q[:, i * _T:(i + 1) * _T]


def kernel(z, codebook):
    return pl.pallas_call(
        _vq_body,
        grid=(_B // _BB,),
        in_specs=[
            pl.BlockSpec((_BB, _D, _T), lambda b: (b, 0, 0)),
            pl.BlockSpec((_K, _D), lambda b: (0, 0)),
        ],
        out_specs=pl.BlockSpec((_BB, _D, _T), lambda b: (b, 0, 0)),
        out_shape=jax.ShapeDtypeStruct((_B, _D, _T), jnp.float32),
        scratch_shapes=[
            pltpu.VMEM((1, _K), jnp.float32),
            pltpu.VMEM((_K, 3 * _D), jnp.bfloat16),
        ],
    )(z, codebook)


# speedup vs baseline: 1.4257x; 1.0145x over previous
import jax
import jax.numpy as jnp
from jax.experimental import pallas as pl
from jax.experimental.pallas import tpu as pltpu

_B, _D, _T = 16, 64, 576
_K = 1024
_BB = 8
_R = (_BB // 2) * _T


def _vq_body(z_ref, cb_ref, out_ref, cc_s, cs_s):
    cb = cb_ref[...]

    @pl.when(pl.program_id(0) == 0)
    def _prep():
        cc_s[0] = jnp.sum(cb * cb, axis=1)
        c1 = cb.astype(jnp.bfloat16)
        e1 = cb - c1.astype(jnp.float32)
        c2 = e1.astype(jnp.bfloat16)
        e2 = e1 - c2.astype(jnp.float32)
        cs_s[:, 0:_D] = c1
        cs_s[:, _D:2 * _D] = c2
        cs_s[:, 2 * _D:3 * _D] = e2.astype(jnp.bfloat16)

    zb = jnp.transpose(z_ref[...], (0, 2, 1)).reshape(_BB * _T, _D)
    m2 = jax.lax.dot_general(zb, cb, (((1,), (1,)), ((), ())))
    zz = jnp.sum(zb * zb, axis=1, keepdims=True)
    dists = (zz - 2.0 * m2) + cc_s[0][None, :]
    idxs = jnp.argmin(dists, axis=1).astype(jnp.int32)
    kio = jax.lax.broadcasted_iota(jnp.int32, (_BB * _T, _K), 1)
    onehot = (kio == idxs[:, None]).astype(jnp.bfloat16)

    qs = jax.lax.dot_general(
        cs_s[...], onehot, (((0,), (1,)), ((), ())),
        preferred_element_type=jnp.float32)
    q = (qs[0:_D] + qs[_D:2 * _D]) + qs[2 * _D:3 * _D]
    for i in range(_BB):
        out_ref[i] = q[:, i * _T:(i + 1) * _T]


def kernel(z, codebook):
    return pl.pallas_call(
        _vq_body,
        grid=(_B // _BB,),
        in_specs=[
            pl.BlockSpec((_BB, _D, _T), lambda b: (b, 0, 0)),
            pl.BlockSpec((_K, _D), lambda b: (0, 0)),
        ],
        out_specs=pl.BlockSpec((_BB, _D, _T), lambda b: (b, 0, 0)),
        out_shape=jax.ShapeDtypeStruct((_B, _D, _T), jnp.float32),
        scratch_shapes=[
            pltpu.VMEM((1, _K), jnp.float32),
            pltpu.VMEM((_K, 3 * _D), jnp.bfloat16),
        ],
    )(z, codebook)
